# Initial kernel scaffold; baseline (speedup 1.0000x reference)
import jax
import jax.numpy as jnp
from jax.experimental import pallas as pl


def kernel(x, edge_index, edge_weight, attention, W_z, b_z, W_r, b_r, W_h, b_h, Lz_W, Lz_b, Lr_W, Lr_b, Lh_W, Lh_b, lin_W, lin_b, lin2_W, lin2_b):
    raise NotImplementedError("write your pallas kernel here")


# speedup vs baseline: 31.6918x; 31.6918x over previous
import functools

import jax
import jax.numpy as jnp
from jax import lax
from jax.experimental import pallas as pl
from jax.experimental.pallas import tpu as pltpu

B, N, F, T = 2, 10000, 16, 12
OUT, TARGETS = 32, 10
C = F * T
RB = 1000
NB = N // RB


def _stage_d_body(xs_ref, u_ref, dinv_ref, lin2_ref, wz_ref, bz_ref, wh_ref,
                  bh_ref, p_ref, linw_ref, linb_ref, lin2b_ref, out_ref):
    i = pl.program_id(1)
    xs = xs_ref[0]
    u = u_ref[0]
    dv = dinv_ref[...]
    y = dv * (xs + u)
    z = jax.nn.sigmoid(
        jnp.dot(y, wz_ref[...], preferred_element_type=jnp.float32) + bz_ref[...])
    ht = jnp.tanh(
        jnp.dot(y, wh_ref[...], preferred_element_type=jnp.float32) + bh_ref[...])
    g = (1.0 - z) * ht
    hacc = jnp.dot(g, p_ref[...], preferred_element_type=jnp.float32)
    rh = jax.nn.relu(hacc)
    h1 = jnp.dot(rh, linw_ref[...], preferred_element_type=jnp.float32) + linb_ref[...]
    contrib = lax.dot_general(lin2_ref[...], h1, (((0,), (0,)), ((), ())),
                              preferred_element_type=jnp.float32)

    @pl.when(i == 0)
    def _():
        out_ref[0] = jnp.broadcast_to(lin2b_ref[...], (TARGETS, T))

    out_ref[0] += contrib


def _stage_d(Xs, U, dinv, lin2_W, WzBD, bzBD, WhBD, bhBD, P, lin_W, lin_b,
             lin2_b, interpret=False):
    grid = (B, NB)
    return pl.pallas_call(
        _stage_d_body,
        grid=grid,
        in_specs=[
            pl.BlockSpec((1, RB, C), lambda b, i: (b, i, 0)),
            pl.BlockSpec((1, RB, C), lambda b, i: (b, i, 0)),
            pl.BlockSpec((RB, 1), lambda b, i: (i, 0)),
            pl.BlockSpec((RB, TARGETS), lambda b, i: (i, 0)),
            pl.BlockSpec((C, T * OUT), lambda b, i: (0, 0)),
            pl.BlockSpec((1, T * OUT), lambda b, i: (0, 0)),
            pl.BlockSpec((C, T * OUT), lambda b, i: (0, 0)),
            pl.BlockSpec((1, T * OUT), lambda b, i: (0, 0)),
            pl.BlockSpec((T * OUT, OUT), lambda b, i: (0, 0)),
            pl.BlockSpec((OUT, T), lambda b, i: (0, 0)),
            pl.BlockSpec((1, T), lambda b, i: (0, 0)),
            pl.BlockSpec((TARGETS, 1), lambda b, i: (0, 0)),
        ],
        out_specs=pl.BlockSpec((1, TARGETS, T), lambda b, i: (b, 0, 0)),
        out_shape=jax.ShapeDtypeStruct((B, TARGETS, T), jnp.float32),
        interpret=interpret,
    )(Xs, U, dinv, lin2_W, WzBD, bzBD, WhBD, bhBD, P, lin_W, lin_b, lin2_b)


def kernel(x, edge_index, edge_weight, attention, W_z, b_z, W_r, b_r, W_h, b_h,
           Lz_W, Lz_b, Lr_W, Lr_b, Lh_W, Lh_b, lin_W, lin_b, lin2_W, lin2_b):
    src, dst = edge_index[0], edge_index[1]
    xf = x.reshape(B, N, C)

    deg0 = jnp.zeros((N,), jnp.float32).at[dst].add(edge_weight)
    dinv = lax.rsqrt(deg0 + 1.0)
    Xs = dinv[None, :, None] * xf
    U = jnp.zeros_like(Xs).at[:, dst, :].add(
        edge_weight[None, :, None] * Xs[:, src, :])

    probs = jax.nn.softmax(attention)
    Wz2 = W_z @ Lz_W[:OUT]
    bz2 = b_z @ Lz_W[:OUT] + Lz_b
    Wh2 = W_h @ Lh_W[:OUT]
    bh2 = b_h @ Lh_W[:OUT] + Lh_b
    fi = jnp.arange(C) // T
    ti = jnp.arange(C) % T
    pi = jnp.arange(T * OUT) // OUT
    oi = jnp.arange(T * OUT) % OUT
    WzBD = Wz2[fi[:, None], oi[None, :]] * (ti[:, None] == pi[None, :])
    WhBD = Wh2[fi[:, None], oi[None, :]] * (ti[:, None] == pi[None, :])
    bzBD = jnp.tile(bz2, T)[None, :]
    bhBD = jnp.tile(bh2, T)[None, :]
    P = jnp.kron(probs[:, None], jnp.eye(OUT, dtype=jnp.float32))

    return _stage_d(Xs, U, dinv[:, None], lin2_W, WzBD, bzBD, WhBD, bhBD, P,
                    lin_W, lin_b[None, :], lin2_b[:, None])


# speedup vs baseline: 727.5392x; 22.9567x over previous
import functools

import jax
import jax.numpy as jnp
from jax import lax
from jax.experimental import pallas as pl
from jax.experimental.pallas import tpu as pltpu
from jax.experimental.pallas import tpu_sc as plsc

B, N, F, T = 2, 10000, 16, 12
OUT, TARGETS = 32, 10
C = F * T
CH = C // 2
RB = 1000
NB = N // RB

NC, NS = 2, 16
G = 80
NPAD = 10240
RPT = NPAD // NS


def _lane_bcast(v, i):
    idx = jnp.full((16, 1), i, jnp.int32)
    dnums = lax.GatherDimensionNumbers(
        offset_dims=(), collapsed_slice_dims=(0,), start_index_map=(0,))
    return lax.gather(v, idx, dnums, (1,),
                      mode=lax.GatherScatterMode.PROMISE_IN_BOUNDS)


def _deg_body(dste_hbm, ew_hbm, z_hbm, degp_hbm, dstv, eww, wrows, wtmp, acc,
              sem):
    c = lax.axis_index("c")
    s = lax.axis_index("s")
    ept = dste_hbm.shape[0] // (NC * NS)
    nchunks = ept // G

    pltpu.sync_copy(z_hbm.at[pl.ds(s * RPT, RPT)], acc.at[pl.ds(s * RPT, RPT)])
    plsc.subcore_barrier()

    def chunk(k, _):
        eb = (c * NS + s) * ept + k * G
        pltpu.sync_copy(dste_hbm.at[pl.ds(eb, G)], dstv)
        pltpu.sync_copy(ew_hbm.at[pl.ds(eb, G)], eww)
        for g in range(G // 16):
            w16 = eww[pl.ds(g * 16, 16)]

            def edge(e2, _):
                wtmp[pl.ds(0, 16)] = _lane_bcast(w16, e2)
                wrows[g * 16 + e2, pl.ds(0, 16)] = wtmp[pl.ds(0, 16)]
                return 0
            lax.fori_loop(0, 16, edge, 0)
        pltpu.sync_copy(wrows, acc.at[dstv], add=True)
        return 0
    lax.fori_loop(0, nchunks, chunk, 0)
    plsc.subcore_barrier()
    pltpu.sync_copy(acc.at[pl.ds(s * RPT, RPT)],
                    degp_hbm.at[c, pl.ds(s * RPT, RPT)])


def _stage_a(dst, ew):
    mesh = plsc.VectorSubcoreMesh(core_axis_name="c", subcore_axis_name="s")
    f = functools.partial(
        pl.kernel,
        out_type=jax.ShapeDtypeStruct((NC, NPAD, 32), jnp.float32),
        mesh=mesh,
        compiler_params=pltpu.CompilerParams(use_tc_tiling_on_sc=False),
        scratch_types=[
            pltpu.VMEM((G,), jnp.int32),
            pltpu.VMEM((G,), jnp.float32),
            pltpu.VMEM((G, 32), jnp.float32),
            pltpu.VMEM((16,), jnp.float32),
            pltpu.VMEM_SHARED((NPAD, 32), jnp.float32),
            pltpu.SemaphoreType.DMA,
        ],
    )(_deg_body)
    return f(dst, ew, jnp.zeros((NPAD, 32), jnp.float32))


def _prescale_body(degp_ref, x_ref, dinv_ref, xsa_ref, xsb_ref):
    deg = degp_ref[0, :, 0:1] + degp_ref[1, :, 0:1] + 1.0
    dv = lax.rsqrt(deg)
    dinv_ref[...] = dv
    xf = x_ref[...].reshape(B, RB, C)
    xs = dv[None, :, :] * xf
    xsa_ref[...] = xs[:, :, :CH]
    xsb_ref[...] = xs[:, :, CH:]


def _stage_b(degp, x4, interpret=False):
    return pl.pallas_call(
        _prescale_body,
        grid=(NB,),
        in_specs=[
            pl.BlockSpec((2, RB, 32), lambda i: (0, i, 0)),
            pl.BlockSpec((B, RB, F, T), lambda i: (0, i, 0, 0)),
        ],
        out_specs=[
            pl.BlockSpec((RB, 1), lambda i: (i, 0)),
            pl.BlockSpec((B, RB, CH), lambda i: (0, i, 0)),
            pl.BlockSpec((B, RB, CH), lambda i: (0, i, 0)),
        ],
        out_shape=[
            jax.ShapeDtypeStruct((N, 1), jnp.float32),
            jax.ShapeDtypeStruct((B, N, CH), jnp.float32),
            jax.ShapeDtypeStruct((B, N, CH), jnp.float32),
        ],
        interpret=interpret,
    )(degp, x4)


def _mp_body(xsa_hbm, xsb_hbm, srce_hbm, dste_hbm, ew_hbm, z_hbm,
             ua_hbm, ub_hbm, srcv, dstv, eww, rows, acc, sem):
    c = lax.axis_index("c")
    s = lax.axis_index("s")
    ept = srce_hbm.shape[0] // NS
    nchunks = ept // G
    coff = (c * N).astype(jnp.int32)

    for xs_hbm, u_hbm in ((xsa_hbm, ua_hbm), (xsb_hbm, ub_hbm)):
        pltpu.sync_copy(z_hbm.at[pl.ds(s * RPT, RPT)],
                        acc.at[pl.ds(s * RPT, RPT)])
        plsc.subcore_barrier()

        def chunk(k, _):
            eb = s * ept + k * G
            pltpu.sync_copy(srce_hbm.at[pl.ds(eb, G)], srcv)
            pltpu.sync_copy(dste_hbm.at[pl.ds(eb, G)], dstv)
            pltpu.sync_copy(ew_hbm.at[pl.ds(eb, G)], eww)
            for g in range(G // 16):
                srcv[pl.ds(g * 16, 16)] = srcv[pl.ds(g * 16, 16)] + jnp.full(
                    (16,), coff, jnp.int32)
            pltpu.async_copy(xs_hbm.at[srcv], rows, sem).wait()

            for g in range(G // 16):
                w16 = eww[pl.ds(g * 16, 16)]

                def edge(e2, _):
                    wb = _lane_bcast(w16, e2)
                    e = g * 16 + e2
                    for j in range(CH // 16):
                        rows[e, pl.ds(j * 16, 16)] = (
                            rows[e, pl.ds(j * 16, 16)] * wb)
                    return 0
                lax.fori_loop(0, 16, edge, 0)
            pltpu.sync_copy(rows, acc.at[dstv], add=True)
            return 0
        lax.fori_loop(0, nchunks, chunk, 0)
        plsc.subcore_barrier()

        @pl.when(s < NS - 1)
        def _():
            pltpu.sync_copy(acc.at[pl.ds(s * RPT, RPT)],
                            u_hbm.at[pl.ds(coff + s * RPT, RPT)])

        @pl.when(s == NS - 1)
        def _():
            last = N - (NS - 1) * RPT
            pltpu.sync_copy(acc.at[pl.ds((NS - 1) * RPT, last)],
                            u_hbm.at[pl.ds(coff + (NS - 1) * RPT, last)])


def _stage_c(XsA2, XsB2, src, dst, ew):
    mesh = plsc.VectorSubcoreMesh(core_axis_name="c", subcore_axis_name="s")
    f = functools.partial(
        pl.kernel,
        out_type=[
            jax.ShapeDtypeStruct((B * N, CH), jnp.float32),
            jax.ShapeDtypeStruct((B * N, CH), jnp.float32),
        ],
        mesh=mesh,
        compiler_params=pltpu.CompilerParams(use_tc_tiling_on_sc=False),
        scratch_types=[
            pltpu.VMEM((G,), jnp.int32),
            pltpu.VMEM((G,), jnp.int32),
            pltpu.VMEM((G,), jnp.float32),
            pltpu.VMEM((G, CH), jnp.float32),
            pltpu.VMEM_SHARED((NPAD, CH), jnp.float32),
            pltpu.SemaphoreType.DMA,
        ],
    )(_mp_body)
    return f(XsA2, XsB2, src, dst, ew, jnp.zeros((NPAD, CH), jnp.float32))


def _stage_d_body(xsa_ref, xsb_ref, ua_ref, ub_ref, dinv_ref, lin2_ref,
                  wz_ref, bz_ref, wh_ref, bh_ref, p_ref, linw_ref, linb_ref,
                  lin2b_ref, out_ref):
    i = pl.program_id(1)
    dv = dinv_ref[...]
    ya = dv * (xsa_ref[0] + ua_ref[0])
    yb = dv * (xsb_ref[0] + ub_ref[0])
    wz = wz_ref[...]
    wh = wh_ref[...]
    z = jax.nn.sigmoid(
        jnp.dot(ya, wz[:CH], preferred_element_type=jnp.float32)
        + jnp.dot(yb, wz[CH:], preferred_element_type=jnp.float32)
        + bz_ref[...])
    ht = jnp.tanh(
        jnp.dot(ya, wh[:CH], preferred_element_type=jnp.float32)
        + jnp.dot(yb, wh[CH:], preferred_element_type=jnp.float32)
        + bh_ref[...])
    g = (1.0 - z) * ht
    hacc = jnp.dot(g, p_ref[...], preferred_element_type=jnp.float32)
    rh = jax.nn.relu(hacc)
    h1 = jnp.dot(rh, linw_ref[...],
                 preferred_element_type=jnp.float32) + linb_ref[...]
    contrib = lax.dot_general(lin2_ref[...], h1, (((0,), (0,)), ((), ())),
                              preferred_element_type=jnp.float32)

    @pl.when(i == 0)
    def _():
        out_ref[0] = jnp.broadcast_to(lin2b_ref[...], (TARGETS, T))

    out_ref[0] += contrib


def _stage_d(XsA, XsB, UA, UB, dinv, lin2_W, WzBD, bzBD, WhBD, bhBD, P,
             lin_W, lin_b, lin2_b, interpret=False):
    grid = (B, NB)
    return pl.pallas_call(
        _stage_d_body,
        grid=grid,
        in_specs=[
            pl.BlockSpec((1, RB, CH), lambda b, i: (b, i, 0)),
            pl.BlockSpec((1, RB, CH), lambda b, i: (b, i, 0)),
            pl.BlockSpec((1, RB, CH), lambda b, i: (b, i, 0)),
            pl.BlockSpec((1, RB, CH), lambda b, i: (b, i, 0)),
            pl.BlockSpec((RB, 1), lambda b, i: (i, 0)),
            pl.BlockSpec((RB, TARGETS), lambda b, i: (i, 0)),
            pl.BlockSpec((C, T * OUT), lambda b, i: (0, 0)),
            pl.BlockSpec((1, T * OUT), lambda b, i: (0, 0)),
            pl.BlockSpec((C, T * OUT), lambda b, i: (0, 0)),
            pl.BlockSpec((1, T * OUT), lambda b, i: (0, 0)),
            pl.BlockSpec((T * OUT, OUT), lambda b, i: (0, 0)),
            pl.BlockSpec((OUT, T), lambda b, i: (0, 0)),
            pl.BlockSpec((1, T), lambda b, i: (0, 0)),
            pl.BlockSpec((TARGETS, 1), lambda b, i: (0, 0)),
        ],
        out_specs=pl.BlockSpec((1, TARGETS, T), lambda b, i: (b, 0, 0)),
        out_shape=jax.ShapeDtypeStruct((B, TARGETS, T), jnp.float32),
        interpret=interpret,
    )(XsA, XsB, UA, UB, dinv, lin2_W, WzBD, bzBD, WhBD, bhBD, P, lin_W,
      lin_b, lin2_b)


def kernel(x, edge_index, edge_weight, attention, W_z, b_z, W_r, b_r, W_h, b_h,
           Lz_W, Lz_b, Lr_W, Lr_b, Lh_W, Lh_b, lin_W, lin_b, lin2_W, lin2_b):
    src, dst = edge_index[0], edge_index[1]

    degp = _stage_a(dst, edge_weight)

    dinv, XsA, XsB = _stage_b(degp, x)

    UA, UB = _stage_c(XsA.reshape(B * N, CH), XsB.reshape(B * N, CH),
                      src, dst, edge_weight)

    probs = jax.nn.softmax(attention)
    Wz2 = W_z @ Lz_W[:OUT]
    bz2 = b_z @ Lz_W[:OUT] + Lz_b
    Wh2 = W_h @ Lh_W[:OUT]
    bh2 = b_h @ Lh_W[:OUT] + Lh_b
    fi = jnp.arange(C) // T
    ti = jnp.arange(C) % T
    pi = jnp.arange(T * OUT) // OUT
    oi = jnp.arange(T * OUT) % OUT
    WzBD = Wz2[fi[:, None], oi[None, :]] * (ti[:, None] == pi[None, :])
    WhBD = Wh2[fi[:, None], oi[None, :]] * (ti[:, None] == pi[None, :])
    bzBD = jnp.tile(bz2, T)[None, :]
    bhBD = jnp.tile(bh2, T)[None, :]
    P = jnp.kron(probs[:, None], jnp.eye(OUT, dtype=jnp.float32))

    return _stage_d(XsA, XsB, UA.reshape(B, N, CH), UB.reshape(B, N, CH),
                    dinv, lin2_W, WzBD, bzBD, WhBD, bhBD, P, lin_W,
                    lin_b[None, :], lin2_b[:, None])


# speedup vs baseline: 1009.0499x; 1.3869x over previous
import functools

import jax
import jax.numpy as jnp
from jax import lax
from jax.experimental import pallas as pl
from jax.experimental.pallas import tpu as pltpu
from jax.experimental.pallas import tpu_sc as plsc

B, N, F, T = 2, 10000, 16, 12
OUT, TARGETS = 32, 10
C = F * T
CH = C // 2
RB = 1000
NB = N // RB

NC, NS = 2, 16
G = 80
NPAD = 10240
RPT = NPAD // NS


def _lane_bcast(v, i):
    idx = jnp.full((16, 1), i, jnp.int32)
    dnums = lax.GatherDimensionNumbers(
        offset_dims=(), collapsed_slice_dims=(0,), start_index_map=(0,))
    return lax.gather(v, idx, dnums, (1,),
                      mode=lax.GatherScatterMode.PROMISE_IN_BOUNDS)


def _deg_body(dste_hbm, ew_hbm, z_hbm, degp_hbm, dstv, eww, wrows, wtmp, acc,
              sem):
    c = lax.axis_index("c")
    s = lax.axis_index("s")
    ept = dste_hbm.shape[0] // (NC * NS)
    nchunks = ept // G

    pltpu.sync_copy(z_hbm.at[pl.ds(s * RPT, RPT)], acc.at[pl.ds(s * RPT, RPT)])
    plsc.subcore_barrier()

    def chunk(k, _):
        eb = (c * NS + s) * ept + k * G
        pltpu.sync_copy(dste_hbm.at[pl.ds(eb, G)], dstv)
        pltpu.sync_copy(ew_hbm.at[pl.ds(eb, G)], eww)
        for g in range(G // 16):
            w16 = eww[pl.ds(g * 16, 16)]

            def edge(e2, _):
                wtmp[pl.ds(0, 16)] = _lane_bcast(w16, e2)
                wrows[g * 16 + e2, pl.ds(0, 16)] = wtmp[pl.ds(0, 16)]
                return 0
            lax.fori_loop(0, 16, edge, 0)
        pltpu.sync_copy(wrows, acc.at[dstv], add=True)
        return 0
    lax.fori_loop(0, nchunks, chunk, 0)
    plsc.subcore_barrier()
    pltpu.sync_copy(acc.at[pl.ds(s * RPT, RPT)],
                    degp_hbm.at[c, pl.ds(s * RPT, RPT)])


def _stage_a(dst, ew):
    mesh = plsc.VectorSubcoreMesh(core_axis_name="c", subcore_axis_name="s")
    f = functools.partial(
        pl.kernel,
        out_type=jax.ShapeDtypeStruct((NC, NPAD, 32), jnp.float32),
        mesh=mesh,
        compiler_params=pltpu.CompilerParams(use_tc_tiling_on_sc=False),
        scratch_types=[
            pltpu.VMEM((G,), jnp.int32),
            pltpu.VMEM((G,), jnp.float32),
            pltpu.VMEM((G, 32), jnp.float32),
            pltpu.VMEM((16,), jnp.float32),
            pltpu.VMEM_SHARED((NPAD, 32), jnp.float32),
            pltpu.SemaphoreType.DMA,
        ],
    )(_deg_body)
    return f(dst, ew, jnp.zeros((NPAD, 32), jnp.float32))


def _prescale_body(degp_ref, x_ref, dinv_ref, xsa_ref, xsb_ref):
    deg = degp_ref[0, :, 0:1] + degp_ref[1, :, 0:1] + 1.0
    dv = lax.rsqrt(deg)
    dinv_ref[...] = dv
    xf = x_ref[...].reshape(B, RB, C)
    xs = dv[None, :, :] * xf
    xsa_ref[...] = xs[:, :, :CH]
    xsb_ref[...] = xs[:, :, CH:]


def _stage_b(degp, x4, interpret=False):
    return pl.pallas_call(
        _prescale_body,
        grid=(NB,),
        in_specs=[
            pl.BlockSpec((2, RB, 32), lambda i: (0, i, 0)),
            pl.BlockSpec((B, RB, F, T), lambda i: (0, i, 0, 0)),
        ],
        out_specs=[
            pl.BlockSpec((RB, 1), lambda i: (i, 0)),
            pl.BlockSpec((B, RB, CH), lambda i: (0, i, 0)),
            pl.BlockSpec((B, RB, CH), lambda i: (0, i, 0)),
        ],
        out_shape=[
            jax.ShapeDtypeStruct((N, 1), jnp.float32),
            jax.ShapeDtypeStruct((B, N, CH), jnp.float32),
            jax.ShapeDtypeStruct((B, N, CH), jnp.float32),
        ],
        interpret=interpret,
    )(degp, x4)


def _mp_body(xsa_hbm, xsb_hbm, srce_hbm, dste_hbm, ew_hbm, z_hbm,
             ua_hbm, ub_hbm,
             srcv0, dstv0, eww0, rows0, srcv1, dstv1, eww1, rows1,
             acc, semg0, semg1, sems0, sems1):
    c = lax.axis_index("c")
    s = lax.axis_index("s")
    ept = srce_hbm.shape[0] // NS
    nchunks = ept // G
    npair = nchunks // 2
    coff = (c * N).astype(jnp.int32)
    bufs = ((srcv0, dstv0, eww0, rows0, semg0, sems0),
            (srcv1, dstv1, eww1, rows1, semg1, sems1))

    def stage(k, srcv, dstv, eww):
        eb = s * ept + k * G
        pltpu.sync_copy(srce_hbm.at[pl.ds(eb, G)], srcv)
        pltpu.sync_copy(dste_hbm.at[pl.ds(eb, G)], dstv)
        pltpu.sync_copy(ew_hbm.at[pl.ds(eb, G)], eww)
        for g in range(G // 16):
            srcv[pl.ds(g * 16, 16)] = srcv[pl.ds(g * 16, 16)] + jnp.full(
                (16,), coff, jnp.int32)

    def scale(rows, eww):
        def grp(g, _):
            w16 = eww[pl.ds(g * 16, 16)]
            for e2 in range(16):
                wb = _lane_bcast(w16, e2)
                e = g * 16 + e2
                for j in range(CH // 16):
                    rows[e, pl.ds(j * 16, 16)] = (
                        rows[e, pl.ds(j * 16, 16)] * wb)
            return 0
        lax.fori_loop(0, G // 16, grp, 0)

    def drain(rows, sem):
        pltpu.make_async_copy(z_hbm.at[pl.ds(0, G)], rows, sem).wait()

    for xs_hbm, u_hbm in ((xsa_hbm, ua_hbm), (xsb_hbm, ub_hbm)):
        pltpu.sync_copy(z_hbm.at[pl.ds(s * RPT, RPT)],
                        acc.at[pl.ds(s * RPT, RPT)])
        plsc.subcore_barrier()

        stage(0, srcv0, dstv0, eww0)
        pltpu.async_copy(xs_hbm.at[srcv0], rows0, semg0)

        def pair(k2, _):
            for b in (0, 1):
                k = 2 * k2 + b
                srcv_c, dstv_c, eww_c, rows_c, semg_c, sems_c = bufs[b]
                srcv_n, dstv_n, eww_n, rows_n, semg_n, sems_n = bufs[1 - b]

                @pl.when(k + 1 < nchunks)
                def _():
                    @pl.when(k >= 1)
                    def _():
                        drain(rows_n, sems_n)
                    stage(k + 1, srcv_n, dstv_n, eww_n)
                    pltpu.async_copy(xs_hbm.at[srcv_n], rows_n, semg_n)

                drain(rows_c, semg_c)
                scale(rows_c, eww_c)
                pltpu.async_copy(rows_c, acc.at[dstv_c], sems_c, add=True)
            return 0
        lax.fori_loop(0, npair, pair, 0)
        drain(rows0, sems0)
        drain(rows1, sems1)
        plsc.subcore_barrier()

        @pl.when(s < NS - 1)
        def _():
            pltpu.sync_copy(acc.at[pl.ds(s * RPT, RPT)],
                            u_hbm.at[pl.ds(coff + s * RPT, RPT)])

        @pl.when(s == NS - 1)
        def _():
            last = N - (NS - 1) * RPT
            pltpu.sync_copy(acc.at[pl.ds((NS - 1) * RPT, last)],
                            u_hbm.at[pl.ds(coff + (NS - 1) * RPT, last)])


def _stage_c(XsA2, XsB2, src, dst, ew):
    mesh = plsc.VectorSubcoreMesh(core_axis_name="c", subcore_axis_name="s")
    f = functools.partial(
        pl.kernel,
        out_type=[
            jax.ShapeDtypeStruct((B * N, CH), jnp.float32),
            jax.ShapeDtypeStruct((B * N, CH), jnp.float32),
        ],
        mesh=mesh,
        compiler_params=pltpu.CompilerParams(use_tc_tiling_on_sc=False),
        scratch_types=[
            pltpu.VMEM((G,), jnp.int32),
            pltpu.VMEM((G,), jnp.int32),
            pltpu.VMEM((G,), jnp.float32),
            pltpu.VMEM((G, CH), jnp.float32),
            pltpu.VMEM((G,), jnp.int32),
            pltpu.VMEM((G,), jnp.int32),
            pltpu.VMEM((G,), jnp.float32),
            pltpu.VMEM((G, CH), jnp.float32),
            pltpu.VMEM_SHARED((NPAD, CH), jnp.float32),
            pltpu.SemaphoreType.DMA,
            pltpu.SemaphoreType.DMA,
            pltpu.SemaphoreType.DMA,
            pltpu.SemaphoreType.DMA,
        ],
    )(_mp_body)
    return f(XsA2, XsB2, src, dst, ew, jnp.zeros((NPAD, CH), jnp.float32))


def _stage_d_body(xsa_ref, xsb_ref, ua_ref, ub_ref, dinv_ref, lin2_ref,
                  wz_ref, bz_ref, wh_ref, bh_ref, p_ref, linw_ref, linb_ref,
                  lin2b_ref, out_ref):
    i = pl.program_id(1)
    dv = dinv_ref[...]
    ya = dv * (xsa_ref[0] + ua_ref[0])
    yb = dv * (xsb_ref[0] + ub_ref[0])
    wz = wz_ref[...]
    wh = wh_ref[...]
    z = jax.nn.sigmoid(
        jnp.dot(ya, wz[:CH], preferred_element_type=jnp.float32)
        + jnp.dot(yb, wz[CH:], preferred_element_type=jnp.float32)
        + bz_ref[...])
    ht = jnp.tanh(
        jnp.dot(ya, wh[:CH], preferred_element_type=jnp.float32)
        + jnp.dot(yb, wh[CH:], preferred_element_type=jnp.float32)
        + bh_ref[...])
    g = (1.0 - z) * ht
    hacc = jnp.dot(g, p_ref[...], preferred_element_type=jnp.float32)
    rh = jax.nn.relu(hacc)
    h1 = jnp.dot(rh, linw_ref[...],
                 preferred_element_type=jnp.float32) + linb_ref[...]
    contrib = lax.dot_general(lin2_ref[...], h1, (((0,), (0,)), ((), ())),
                              preferred_element_type=jnp.float32)

    @pl.when(i == 0)
    def _():
        out_ref[0] = jnp.broadcast_to(lin2b_ref[...], (TARGETS, T))

    out_ref[0] += contrib


def _stage_d(XsA, XsB, UA, UB, dinv, lin2_W, WzBD, bzBD, WhBD, bhBD, P,
             lin_W, lin_b, lin2_b, interpret=False):
    grid = (B, NB)
    return pl.pallas_call(
        _stage_d_body,
        grid=grid,
        in_specs=[
            pl.BlockSpec((1, RB, CH), lambda b, i: (b, i, 0)),
            pl.BlockSpec((1, RB, CH), lambda b, i: (b, i, 0)),
            pl.BlockSpec((1, RB, CH), lambda b, i: (b, i, 0)),
            pl.BlockSpec((1, RB, CH), lambda b, i: (b, i, 0)),
            pl.BlockSpec((RB, 1), lambda b, i: (i, 0)),
            pl.BlockSpec((RB, TARGETS), lambda b, i: (i, 0)),
            pl.BlockSpec((C, T * OUT), lambda b, i: (0, 0)),
            pl.BlockSpec((1, T * OUT), lambda b, i: (0, 0)),
            pl.BlockSpec((C, T * OUT), lambda b, i: (0, 0)),
            pl.BlockSpec((1, T * OUT), lambda b, i: (0, 0)),
            pl.BlockSpec((T * OUT, OUT), lambda b, i: (0, 0)),
            pl.BlockSpec((OUT, T), lambda b, i: (0, 0)),
            pl.BlockSpec((1, T), lambda b, i: (0, 0)),
            pl.BlockSpec((TARGETS, 1), lambda b, i: (0, 0)),
        ],
        out_specs=pl.BlockSpec((1, TARGETS, T), lambda b, i: (b, 0, 0)),
        out_shape=jax.ShapeDtypeStruct((B, TARGETS, T), jnp.float32),
        interpret=interpret,
    )(XsA, XsB, UA, UB, dinv, lin2_W, WzBD, bzBD, WhBD, bhBD, P, lin_W,
      lin_b, lin2_b)


def kernel(x, edge_index, edge_weight, attention, W_z, b_z, W_r, b_r, W_h, b_h,
           Lz_W, Lz_b, Lr_W, Lr_b, Lh_W, Lh_b, lin_W, lin_b, lin2_W, lin2_b):
    src, dst = edge_index[0], edge_index[1]

    degp = _stage_a(dst, edge_weight)

    dinv, XsA, XsB = _stage_b(degp, x)

    UA, UB = _stage_c(XsA.reshape(B * N, CH), XsB.reshape(B * N, CH),
                      src, dst, edge_weight)

    probs = jax.nn.softmax(attention)
    Wz2 = W_z @ Lz_W[:OUT]
    bz2 = b_z @ Lz_W[:OUT] + Lz_b
    Wh2 = W_h @ Lh_W[:OUT]
    bh2 = b_h @ Lh_W[:OUT] + Lh_b
    fi = jnp.arange(C) // T
    ti = jnp.arange(C) % T
    pi = jnp.arange(T * OUT) // OUT
    oi = jnp.arange(T * OUT) % OUT
    WzBD = Wz2[fi[:, None], oi[None, :]] * (ti[:, None] == pi[None, :])
    WhBD = Wh2[fi[:, None], oi[None, :]] * (ti[:, None] == pi[None, :])
    bzBD = jnp.tile(bz2, T)[None, :]
    bhBD = jnp.tile(bh2, T)[None, :]
    P = jnp.kron(probs[:, None], jnp.eye(OUT, dtype=jnp.float32))

    return _stage_d(XsA, XsB, UA.reshape(B, N, CH), UB.reshape(B, N, CH),
                    dinv, lin2_W, WzBD, bzBD, WhBD, bhBD, P, lin_W,
                    lin_b[None, :], lin2_b[:, None])


# speedup vs baseline: 1358.5641x; 1.3464x over previous
import functools

import jax
import jax.numpy as jnp
from jax import lax
from jax.experimental import pallas as pl
from jax.experimental.pallas import tpu as pltpu
from jax.experimental.pallas import tpu_sc as plsc

B, N, F, T = 2, 10000, 16, 12
OUT, TARGETS = 32, 10
C = F * T
CH = C // 2
RB = 1000
NB = N // RB

NC, NS = 2, 16
G = 80
NPAD = 10240
HSPLIT = 5
RPT = NPAD // NS


def _lane_bcast(v, i):
    idx = jnp.full((16, 1), i, jnp.int32)
    dnums = lax.GatherDimensionNumbers(
        offset_dims=(), collapsed_slice_dims=(0,), start_index_map=(0,))
    return lax.gather(v, idx, dnums, (1,),
                      mode=lax.GatherScatterMode.PROMISE_IN_BOUNDS)


def _deg_body(dste_hbm, ew_hbm, z_hbm, degp_hbm, dstv, eww, wrows, wtmp, acc,
              sem):
    c = lax.axis_index("c")
    s = lax.axis_index("s")
    ept = dste_hbm.shape[0] // (NC * NS)
    nchunks = ept // G

    pltpu.sync_copy(z_hbm.at[pl.ds(s * RPT, RPT)], acc.at[pl.ds(s * RPT, RPT)])
    plsc.subcore_barrier()

    def chunk(k, _):
        eb = (c * NS + s) * ept + k * G
        pltpu.sync_copy(dste_hbm.at[pl.ds(eb, G)], dstv)
        pltpu.sync_copy(ew_hbm.at[pl.ds(eb, G)], eww)
        for g in range(G // 16):
            w16 = eww[pl.ds(g * 16, 16)]

            def edge(e2, _):
                wtmp[pl.ds(0, 16)] = _lane_bcast(w16, e2)
                wrows[g * 16 + e2, pl.ds(0, 16)] = wtmp[pl.ds(0, 16)]
                return 0
            lax.fori_loop(0, 16, edge, 0)
        pltpu.sync_copy(wrows, acc.at[dstv], add=True)
        return 0
    lax.fori_loop(0, nchunks, chunk, 0)
    plsc.subcore_barrier()
    pltpu.sync_copy(acc.at[pl.ds(s * RPT, RPT)],
                    degp_hbm.at[c, pl.ds(s * RPT, RPT)])


def _stage_a(dst, ew):
    mesh = plsc.VectorSubcoreMesh(core_axis_name="c", subcore_axis_name="s")
    f = functools.partial(
        pl.kernel,
        out_type=jax.ShapeDtypeStruct((NC, NPAD, 32), jnp.float32),
        mesh=mesh,
        compiler_params=pltpu.CompilerParams(use_tc_tiling_on_sc=False),
        scratch_types=[
            pltpu.VMEM((G,), jnp.int32),
            pltpu.VMEM((G,), jnp.float32),
            pltpu.VMEM((G, 32), jnp.float32),
            pltpu.VMEM((16,), jnp.float32),
            pltpu.VMEM_SHARED((NPAD, 32), jnp.float32),
            pltpu.SemaphoreType.DMA,
        ],
    )(_deg_body)
    return f(dst, ew, jnp.zeros((NPAD, 32), jnp.float32))


def _prescale_body(degp_ref, x_ref, dinv_ref, xs_ref):
    deg = degp_ref[0, :, 0:1] + degp_ref[1, :, 0:1] + 1.0
    dv = lax.rsqrt(deg)
    dinv_ref[...] = dv
    xf = x_ref[...].reshape(B, RB, C)
    xs = dv[None, :, :] * xf
    xs_ref[0] = xs[:, :, :CH]
    xs_ref[1] = xs[:, :, CH:]


def _stage_b(degp, x4, interpret=False):
    return pl.pallas_call(
        _prescale_body,
        grid=(NB,),
        in_specs=[
            pl.BlockSpec((2, RB, 32), lambda i: (0, i, 0)),
            pl.BlockSpec((B, RB, F, T), lambda i: (0, i, 0, 0)),
        ],
        out_specs=[
            pl.BlockSpec((RB, 1), lambda i: (i, 0)),
            pl.BlockSpec((2, B, RB, CH), lambda i: (0, 0, i, 0)),
        ],
        out_shape=[
            jax.ShapeDtypeStruct((N, 1), jnp.float32),
            jax.ShapeDtypeStruct((2, B, N, CH), jnp.float32),
        ],
        interpret=interpret,
    )(degp, x4)


def _mp_body(xs_hbm, src_hbm, dst_hbm, ew_hbm, z_hbm, u_hbm,
             srcb, dstb, ewb, rows0, rows1,
             acc, semg0, semg1, sems0, sems1):
    c = lax.axis_index("c")
    s = lax.axis_index("s")
    ept = src_hbm.shape[0] // NS
    nchunks = ept // G
    half = nchunks // HSPLIT
    hw = half * G
    rbufs = ((rows0, semg0, sems0), (rows1, semg1, sems1))

    def scale(rows, j):
        def grp(g, _):
            w16 = ewb[pl.ds(j * G + g * 16, 16)]
            for e2 in range(16):
                wb = _lane_bcast(w16, e2)
                e = g * 16 + e2
                for jj in range(CH // 16):
                    rows[e, pl.ds(jj * 16, 16)] = (
                        rows[e, pl.ds(jj * 16, 16)] * wb)
            return 0
        lax.fori_loop(0, G // 16, grp, 0)

    def drain(rows, sem):
        pltpu.make_async_copy(z_hbm.at[pl.ds(0, G)], rows, sem).wait()

    def phase(p, _):
        roff = p * (B * N) + c * N
        pltpu.sync_copy(z_hbm.at[pl.ds(s * RPT, RPT)],
                        acc.at[pl.ds(s * RPT, RPT)])
        plsc.subcore_barrier()

        def hloop(h, _):
            e0 = s * ept + h * hw
            pltpu.sync_copy(src_hbm.at[pl.ds(e0, hw)], srcb)
            pltpu.sync_copy(dst_hbm.at[pl.ds(e0, hw)], dstb)
            pltpu.sync_copy(ew_hbm.at[pl.ds(e0, hw)], ewb)

            off16 = jnp.full((16,), roff, jnp.int32)

            def adj(r, _):
                srcb[pl.ds(r * 16, 16)] = srcb[pl.ds(r * 16, 16)] + off16
                return 0
            lax.fori_loop(0, hw // 16, adj, 0)

            pltpu.async_copy(xs_hbm.at[srcb.at[pl.ds(0, G)]], rows0, semg0)

            def pair(j2, _):
                for b in (0, 1):
                    j = 2 * j2 + b
                    rows_c, semg_c, sems_c = rbufs[b]
                    rows_n, semg_n, sems_n = rbufs[1 - b]

                    @pl.when(j + 1 < half)
                    def _():
                        @pl.when(j >= 1)
                        def _():
                            drain(rows_n, sems_n)
                        pltpu.async_copy(
                            xs_hbm.at[srcb.at[pl.ds((j + 1) * G, G)]],
                            rows_n, semg_n)

                    drain(rows_c, semg_c)
                    scale(rows_c, j)
                    pltpu.async_copy(rows_c, acc.at[dstb.at[pl.ds(j * G, G)]],
                                     sems_c, add=True)
                return 0
            lax.fori_loop(0, half // 2, pair, 0)
            drain(rows0, sems0)
            drain(rows1, sems1)
            return 0
        lax.fori_loop(0, HSPLIT, hloop, 0)
        plsc.subcore_barrier()

        @pl.when(s < NS - 1)
        def _():
            pltpu.sync_copy(acc.at[pl.ds(s * RPT, RPT)],
                            u_hbm.at[pl.ds(roff + s * RPT, RPT)])

        @pl.when(s == NS - 1)
        def _():
            last = N - (NS - 1) * RPT
            pltpu.sync_copy(acc.at[pl.ds((NS - 1) * RPT, last)],
                            u_hbm.at[pl.ds(roff + (NS - 1) * RPT, last)])
        return 0
    lax.fori_loop(0, 2, phase, 0)


def _stage_c(XsAll, srce, dste, ewe):
    ept = srce.shape[0] // NS
    hw = (ept // G // HSPLIT) * G
    mesh = plsc.VectorSubcoreMesh(core_axis_name="c", subcore_axis_name="s")
    f = functools.partial(
        pl.kernel,
        out_type=jax.ShapeDtypeStruct((2 * B * N, CH), jnp.float32),
        mesh=mesh,
        compiler_params=pltpu.CompilerParams(use_tc_tiling_on_sc=False),
        scratch_types=[
            pltpu.VMEM((hw,), jnp.int32),
            pltpu.VMEM((hw,), jnp.int32),
            pltpu.VMEM((hw,), jnp.float32),
            pltpu.VMEM((G, CH), jnp.float32),
            pltpu.VMEM((G, CH), jnp.float32),
            pltpu.VMEM_SHARED((NPAD, CH), jnp.float32),
            pltpu.SemaphoreType.DMA,
            pltpu.SemaphoreType.DMA,
            pltpu.SemaphoreType.DMA,
            pltpu.SemaphoreType.DMA,
        ],
    )(_mp_body)
    return f(XsAll, srce, dste, ewe, jnp.zeros((NPAD, CH), jnp.float32))


def _stage_d_body(xsa_ref, xsb_ref, ua_ref, ub_ref, dinv_ref, lin2_ref,
                  wz_ref, bz_ref, wh_ref, bh_ref, p_ref, linw_ref, linb_ref,
                  lin2b_ref, out_ref):
    i = pl.program_id(1)
    dv = dinv_ref[...]
    ya = dv * (xsa_ref[0] + ua_ref[0])
    yb = dv * (xsb_ref[0] + ub_ref[0])
    wz = wz_ref[...]
    wh = wh_ref[...]
    z = jax.nn.sigmoid(
        jnp.dot(ya, wz[:CH], preferred_element_type=jnp.float32)
        + jnp.dot(yb, wz[CH:], preferred_element_type=jnp.float32)
        + bz_ref[...])
    ht = jnp.tanh(
        jnp.dot(ya, wh[:CH], preferred_element_type=jnp.float32)
        + jnp.dot(yb, wh[CH:], preferred_element_type=jnp.float32)
        + bh_ref[...])
    g = (1.0 - z) * ht
    hacc = jnp.dot(g, p_ref[...], preferred_element_type=jnp.float32)
    rh = jax.nn.relu(hacc)
    h1 = jnp.dot(rh, linw_ref[...],
                 preferred_element_type=jnp.float32) + linb_ref[...]
    contrib = lax.dot_general(lin2_ref[...], h1, (((0,), (0,)), ((), ())),
                              preferred_element_type=jnp.float32)

    @pl.when(i == 0)
    def _():
        out_ref[0] = jnp.broadcast_to(lin2b_ref[...], (TARGETS, T))

    out_ref[0] += contrib


def _stage_d(XsA, XsB, UA, UB, dinv, lin2_W, WzBD, bzBD, WhBD, bhBD, P,
             lin_W, lin_b, lin2_b, interpret=False):
    grid = (B, NB)
    return pl.pallas_call(
        _stage_d_body,
        grid=grid,
        in_specs=[
            pl.BlockSpec((1, RB, CH), lambda b, i: (b, i, 0)),
            pl.BlockSpec((1, RB, CH), lambda b, i: (b, i, 0)),
            pl.BlockSpec((1, RB, CH), lambda b, i: (b, i, 0)),
            pl.BlockSpec((1, RB, CH), lambda b, i: (b, i, 0)),
            pl.BlockSpec((RB, 1), lambda b, i: (i, 0)),
            pl.BlockSpec((RB, TARGETS), lambda b, i: (i, 0)),
            pl.BlockSpec((C, T * OUT), lambda b, i: (0, 0)),
            pl.BlockSpec((1, T * OUT), lambda b, i: (0, 0)),
            pl.BlockSpec((C, T * OUT), lambda b, i: (0, 0)),
            pl.BlockSpec((1, T * OUT), lambda b, i: (0, 0)),
            pl.BlockSpec((T * OUT, OUT), lambda b, i: (0, 0)),
            pl.BlockSpec((OUT, T), lambda b, i: (0, 0)),
            pl.BlockSpec((1, T), lambda b, i: (0, 0)),
            pl.BlockSpec((TARGETS, 1), lambda b, i: (0, 0)),
        ],
        out_specs=pl.BlockSpec((1, TARGETS, T), lambda b, i: (b, 0, 0)),
        out_shape=jax.ShapeDtypeStruct((B, TARGETS, T), jnp.float32),
        interpret=interpret,
    )(XsA, XsB, UA, UB, dinv, lin2_W, WzBD, bzBD, WhBD, bhBD, P, lin_W,
      lin_b, lin2_b)


def kernel(x, edge_index, edge_weight, attention, W_z, b_z, W_r, b_r, W_h, b_h,
           Lz_W, Lz_b, Lr_W, Lr_b, Lh_W, Lh_b, lin_W, lin_b, lin2_W, lin2_b):
    src, dst = edge_index[0], edge_index[1]

    degp = _stage_a(dst, edge_weight)

    dinv, XsAll = _stage_b(degp, x)

    UAll = _stage_c(XsAll.reshape(2 * B * N, CH), src, dst,
                    edge_weight).reshape(2, B, N, CH)

    probs = jax.nn.softmax(attention)
    Wz2 = W_z @ Lz_W[:OUT]
    bz2 = b_z @ Lz_W[:OUT] + Lz_b
    Wh2 = W_h @ Lh_W[:OUT]
    bh2 = b_h @ Lh_W[:OUT] + Lh_b
    fi = jnp.arange(C) // T
    ti = jnp.arange(C) % T
    pi = jnp.arange(T * OUT) // OUT
    oi = jnp.arange(T * OUT) % OUT
    WzBD = Wz2[fi[:, None], oi[None, :]] * (ti[:, None] == pi[None, :])
    WhBD = Wh2[fi[:, None], oi[None, :]] * (ti[:, None] == pi[None, :])
    bzBD = jnp.tile(bz2, T)[None, :]
    bhBD = jnp.tile(bh2, T)[None, :]
    P = jnp.kron(probs[:, None], jnp.eye(OUT, dtype=jnp.float32))

    return _stage_d(XsAll[0], XsAll[1], UAll[0], UAll[1],
                    dinv, lin2_W, WzBD, bzBD, WhBD, bhBD, P, lin_W,
                    lin_b[None, :], lin2_b[:, None])


# speedup vs baseline: 1580.4963x; 1.1634x over previous
import functools

import jax
import jax.numpy as jnp
from jax import lax
from jax.experimental import pallas as pl
from jax.experimental.pallas import tpu as pltpu
from jax.experimental.pallas import tpu_sc as plsc

B, N, F, T = 2, 10000, 16, 12
OUT, TARGETS = 32, 10
C = F * T
CH = C // 2
RB = 1000
NB = N // RB

NC, NS = 2, 16
G = 80
NPAD = 10240
HSPLIT = 5
RPT = NPAD // NS


def _lane_bcast(v, i):
    idx = jnp.full((16, 1), i, jnp.int32)
    dnums = lax.GatherDimensionNumbers(
        offset_dims=(), collapsed_slice_dims=(0,), start_index_map=(0,))
    return lax.gather(v, idx, dnums, (1,),
                      mode=lax.GatherScatterMode.PROMISE_IN_BOUNDS)


def _deg_body(dste_hbm, ew_hbm, z_hbm, degp_hbm,
              dstb, ewb, wrows0, wrows1, wtmp, acc, sems0, sems1):
    c = lax.axis_index("c")
    s = lax.axis_index("s")
    ept = dste_hbm.shape[0] // (NC * NS)
    nchunks = ept // G
    half = nchunks // HSPLIT
    hw = half * G
    wbufs = ((wrows0, sems0), (wrows1, sems1))

    pltpu.sync_copy(z_hbm.at[pl.ds(s * RPT, RPT)], acc.at[pl.ds(s * RPT, RPT)])
    plsc.subcore_barrier()

    def build(wrows, j):
        def grp(g, _):
            w16 = ewb[pl.ds(j * G + g * 16, 16)]
            for e2 in range(16):
                wtmp[pl.ds(0, 16)] = _lane_bcast(w16, e2)
                wrows[g * 16 + e2, pl.ds(0, 16)] = wtmp[pl.ds(0, 16)]
            return 0
        lax.fori_loop(0, G // 16, grp, 0)

    def drain(wrows, sem):
        pltpu.make_async_copy(z_hbm.at[pl.ds(0, G), pl.ds(0, 32)], wrows,
                              sem).wait()

    def hloop(h, _):
        e0 = (c * NS + s) * ept + h * hw
        pltpu.sync_copy(dste_hbm.at[pl.ds(e0, hw)], dstb)
        pltpu.sync_copy(ew_hbm.at[pl.ds(e0, hw)], ewb)

        def pair(j2, _):
            for b in (0, 1):
                j = 2 * j2 + b
                wrows_c, sems_c = wbufs[b]

                @pl.when(j >= 2)
                def _():
                    drain(wrows_c, sems_c)
                build(wrows_c, j)
                pltpu.async_copy(wrows_c, acc.at[dstb.at[pl.ds(j * G, G)]],
                                 sems_c, add=True)
            return 0
        lax.fori_loop(0, half // 2, pair, 0)
        drain(wrows0, sems0)
        drain(wrows1, sems1)
        return 0
    lax.fori_loop(0, HSPLIT, hloop, 0)
    plsc.subcore_barrier()
    pltpu.sync_copy(acc.at[pl.ds(s * RPT, RPT)],
                    degp_hbm.at[c, pl.ds(s * RPT, RPT)])


def _stage_a(dst, ew):
    ept = dst.shape[0] // (NC * NS)
    hw = (ept // G // HSPLIT) * G
    mesh = plsc.VectorSubcoreMesh(core_axis_name="c", subcore_axis_name="s")
    f = functools.partial(
        pl.kernel,
        out_type=jax.ShapeDtypeStruct((NC, NPAD, 32), jnp.float32),
        mesh=mesh,
        compiler_params=pltpu.CompilerParams(use_tc_tiling_on_sc=False),
        scratch_types=[
            pltpu.VMEM((hw,), jnp.int32),
            pltpu.VMEM((hw,), jnp.float32),
            pltpu.VMEM((G, 32), jnp.float32),
            pltpu.VMEM((G, 32), jnp.float32),
            pltpu.VMEM((16,), jnp.float32),
            pltpu.VMEM_SHARED((NPAD, 32), jnp.float32),
            pltpu.SemaphoreType.DMA,
            pltpu.SemaphoreType.DMA,
        ],
    )(_deg_body)
    return f(dst, ew, jnp.zeros((NPAD, 32), jnp.float32))


def _prescale_body(degp_ref, x_ref, dinv_ref, xs_ref):
    deg = degp_ref[0, :, 0:1] + degp_ref[1, :, 0:1] + 1.0
    dv = lax.rsqrt(deg)
    dinv_ref[...] = dv
    xf = x_ref[...].reshape(B, RB, C)
    xs = dv[None, :, :] * xf
    xs_ref[0] = xs[:, :, :CH]
    xs_ref[1] = xs[:, :, CH:]


def _stage_b(degp, x4, interpret=False):
    return pl.pallas_call(
        _prescale_body,
        grid=(NB,),
        in_specs=[
            pl.BlockSpec((2, RB, 32), lambda i: (0, i, 0)),
            pl.BlockSpec((B, RB, F, T), lambda i: (0, i, 0, 0)),
        ],
        out_specs=[
            pl.BlockSpec((RB, 1), lambda i: (i, 0)),
            pl.BlockSpec((2, B, RB, CH), lambda i: (0, 0, i, 0)),
        ],
        out_shape=[
            jax.ShapeDtypeStruct((N, 1), jnp.float32),
            jax.ShapeDtypeStruct((2, B, N, CH), jnp.float32),
        ],
        interpret=interpret,
    )(degp, x4)


def _mp_body(xs_hbm, src_hbm, dst_hbm, ew_hbm, z_hbm, u_hbm,
             srcb, dstb, ewb, rows0, rows1,
             acc, semg0, semg1, sems0, sems1):
    c = lax.axis_index("c")
    s = lax.axis_index("s")
    ept = src_hbm.shape[0] // NS
    nchunks = ept // G
    half = nchunks // HSPLIT
    hw = half * G
    rbufs = ((rows0, semg0, sems0), (rows1, semg1, sems1))

    def scale(rows, j):
        def grp(g, _):
            w16 = ewb[pl.ds(j * G + g * 16, 16)]
            for e2 in range(16):
                wb = _lane_bcast(w16, e2)
                e = g * 16 + e2
                for jj in range(CH // 16):
                    rows[e, pl.ds(jj * 16, 16)] = (
                        rows[e, pl.ds(jj * 16, 16)] * wb)
            return 0
        lax.fori_loop(0, G // 16, grp, 0)

    def drain(rows, sem):
        pltpu.make_async_copy(z_hbm.at[pl.ds(0, G)], rows, sem).wait()

    def phase(p, _):
        roff = p * (B * N) + c * N
        pltpu.sync_copy(z_hbm.at[pl.ds(s * RPT, RPT)],
                        acc.at[pl.ds(s * RPT, RPT)])
        plsc.subcore_barrier()

        def hloop(h, _):
            e0 = s * ept + h * hw
            pltpu.sync_copy(src_hbm.at[pl.ds(e0, hw)], srcb)
            pltpu.sync_copy(dst_hbm.at[pl.ds(e0, hw)], dstb)
            pltpu.sync_copy(ew_hbm.at[pl.ds(e0, hw)], ewb)

            off16 = jnp.full((16,), roff, jnp.int32)

            def adj(r, _):
                srcb[pl.ds(r * 16, 16)] = srcb[pl.ds(r * 16, 16)] + off16
                return 0
            lax.fori_loop(0, hw // 16, adj, 0)

            pltpu.async_copy(xs_hbm.at[srcb.at[pl.ds(0, G)]], rows0, semg0)

            def pair(j2, _):
                for b in (0, 1):
                    j = 2 * j2 + b
                    rows_c, semg_c, sems_c = rbufs[b]
                    rows_n, semg_n, sems_n = rbufs[1 - b]

                    @pl.when(j + 1 < half)
                    def _():
                        @pl.when(j >= 1)
                        def _():
                            drain(rows_n, sems_n)
                        pltpu.async_copy(
                            xs_hbm.at[srcb.at[pl.ds((j + 1) * G, G)]],
                            rows_n, semg_n)

                    drain(rows_c, semg_c)
                    scale(rows_c, j)
                    pltpu.async_copy(rows_c, acc.at[dstb.at[pl.ds(j * G, G)]],
                                     sems_c, add=True)
                return 0
            lax.fori_loop(0, half // 2, pair, 0)
            drain(rows0, sems0)
            drain(rows1, sems1)
            return 0
        lax.fori_loop(0, HSPLIT, hloop, 0)
        plsc.subcore_barrier()

        @pl.when(s < NS - 1)
        def _():
            pltpu.sync_copy(acc.at[pl.ds(s * RPT, RPT)],
                            u_hbm.at[pl.ds(roff + s * RPT, RPT)])

        @pl.when(s == NS - 1)
        def _():
            last = N - (NS - 1) * RPT
            pltpu.sync_copy(acc.at[pl.ds((NS - 1) * RPT, last)],
                            u_hbm.at[pl.ds(roff + (NS - 1) * RPT, last)])
        return 0
    lax.fori_loop(0, 2, phase, 0)


def _stage_c(XsAll, srce, dste, ewe):
    ept = srce.shape[0] // NS
    hw = (ept // G // HSPLIT) * G
    mesh = plsc.VectorSubcoreMesh(core_axis_name="c", subcore_axis_name="s")
    f = functools.partial(
        pl.kernel,
        out_type=jax.ShapeDtypeStruct((2 * B * N, CH), jnp.float32),
        mesh=mesh,
        compiler_params=pltpu.CompilerParams(use_tc_tiling_on_sc=False),
        scratch_types=[
            pltpu.VMEM((hw,), jnp.int32),
            pltpu.VMEM((hw,), jnp.int32),
            pltpu.VMEM((hw,), jnp.float32),
            pltpu.VMEM((G, CH), jnp.float32),
            pltpu.VMEM((G, CH), jnp.float32),
            pltpu.VMEM_SHARED((NPAD, CH), jnp.float32),
            pltpu.SemaphoreType.DMA,
            pltpu.SemaphoreType.DMA,
            pltpu.SemaphoreType.DMA,
            pltpu.SemaphoreType.DMA,
        ],
    )(_mp_body)
    return f(XsAll, srce, dste, ewe, jnp.zeros((NPAD, CH), jnp.float32))


def _stage_d_body(xsa_ref, xsb_ref, ua_ref, ub_ref, dinv_ref, lin2_ref,
                  wz_ref, bz_ref, wh_ref, bh_ref, p_ref, linw_ref, linb_ref,
                  lin2b_ref, out_ref):
    i = pl.program_id(1)
    dv = dinv_ref[...]
    ya = dv * (xsa_ref[0] + ua_ref[0])
    yb = dv * (xsb_ref[0] + ub_ref[0])
    wz = wz_ref[...]
    wh = wh_ref[...]
    z = jax.nn.sigmoid(
        jnp.dot(ya, wz[:CH], preferred_element_type=jnp.float32)
        + jnp.dot(yb, wz[CH:], preferred_element_type=jnp.float32)
        + bz_ref[...])
    ht = jnp.tanh(
        jnp.dot(ya, wh[:CH], preferred_element_type=jnp.float32)
        + jnp.dot(yb, wh[CH:], preferred_element_type=jnp.float32)
        + bh_ref[...])
    g = (1.0 - z) * ht
    hacc = jnp.dot(g, p_ref[...], preferred_element_type=jnp.float32)
    rh = jax.nn.relu(hacc)
    h1 = jnp.dot(rh, linw_ref[...],
                 preferred_element_type=jnp.float32) + linb_ref[...]
    contrib = lax.dot_general(lin2_ref[...], h1, (((0,), (0,)), ((), ())),
                              preferred_element_type=jnp.float32)

    @pl.when(i == 0)
    def _():
        out_ref[0] = jnp.broadcast_to(lin2b_ref[...], (TARGETS, T))

    out_ref[0] += contrib


def _stage_d(XsA, XsB, UA, UB, dinv, lin2_W, WzBD, bzBD, WhBD, bhBD, P,
             lin_W, lin_b, lin2_b, interpret=False):
    grid = (B, NB)
    return pl.pallas_call(
        _stage_d_body,
        grid=grid,
        in_specs=[
            pl.BlockSpec((1, RB, CH), lambda b, i: (b, i, 0)),
            pl.BlockSpec((1, RB, CH), lambda b, i: (b, i, 0)),
            pl.BlockSpec((1, RB, CH), lambda b, i: (b, i, 0)),
            pl.BlockSpec((1, RB, CH), lambda b, i: (b, i, 0)),
            pl.BlockSpec((RB, 1), lambda b, i: (i, 0)),
            pl.BlockSpec((RB, TARGETS), lambda b, i: (i, 0)),
            pl.BlockSpec((C, T * OUT), lambda b, i: (0, 0)),
            pl.BlockSpec((1, T * OUT), lambda b, i: (0, 0)),
            pl.BlockSpec((C, T * OUT), lambda b, i: (0, 0)),
            pl.BlockSpec((1, T * OUT), lambda b, i: (0, 0)),
            pl.BlockSpec((T * OUT, OUT), lambda b, i: (0, 0)),
            pl.BlockSpec((OUT, T), lambda b, i: (0, 0)),
            pl.BlockSpec((1, T), lambda b, i: (0, 0)),
            pl.BlockSpec((TARGETS, 1), lambda b, i: (0, 0)),
        ],
        out_specs=pl.BlockSpec((1, TARGETS, T), lambda b, i: (b, 0, 0)),
        out_shape=jax.ShapeDtypeStruct((B, TARGETS, T), jnp.float32),
        interpret=interpret,
    )(XsA, XsB, UA, UB, dinv, lin2_W, WzBD, bzBD, WhBD, bhBD, P, lin_W,
      lin_b, lin2_b)


def kernel(x, edge_index, edge_weight, attention, W_z, b_z, W_r, b_r, W_h, b_h,
           Lz_W, Lz_b, Lr_W, Lr_b, Lh_W, Lh_b, lin_W, lin_b, lin2_W, lin2_b):
    src, dst = edge_index[0], edge_index[1]

    degp = _stage_a(dst, edge_weight)

    dinv, XsAll = _stage_b(degp, x)

    UAll = _stage_c(XsAll.reshape(2 * B * N, CH), src, dst,
                    edge_weight).reshape(2, B, N, CH)

    probs = jax.nn.softmax(attention)
    Wz2 = W_z @ Lz_W[:OUT]
    bz2 = b_z @ Lz_W[:OUT] + Lz_b
    Wh2 = W_h @ Lh_W[:OUT]
    bh2 = b_h @ Lh_W[:OUT] + Lh_b
    fi = jnp.arange(C) // T
    ti = jnp.arange(C) % T
    pi = jnp.arange(T * OUT) // OUT
    oi = jnp.arange(T * OUT) % OUT
    WzBD = Wz2[fi[:, None], oi[None, :]] * (ti[:, None] == pi[None, :])
    WhBD = Wh2[fi[:, None], oi[None, :]] * (ti[:, None] == pi[None, :])
    bzBD = jnp.tile(bz2, T)[None, :]
    bhBD = jnp.tile(bh2, T)[None, :]
    P = jnp.kron(probs[:, None], jnp.eye(OUT, dtype=jnp.float32))

    return _stage_d(XsAll[0], XsAll[1], UAll[0], UAll[1],
                    dinv, lin2_W, WzBD, bzBD, WhBD, bhBD, P, lin_W,
                    lin_b[None, :], lin2_b[:, None])


# speedup vs baseline: 1581.1766x; 1.0004x over previous
import functools

import jax
import jax.numpy as jnp
from jax import lax
from jax.experimental import pallas as pl
from jax.experimental.pallas import tpu as pltpu
from jax.experimental.pallas import tpu_sc as plsc

B, N, F, T = 2, 10000, 16, 12
OUT, TARGETS = 32, 10
C = F * T
CH = C // 2
RB = 1000
NB = N // RB

NC, NS = 2, 16
G = 80
NPAD = 10240
HSPLIT = 5
RPT = NPAD // NS


def _lane_bcast(v, i):
    idx = jnp.full((16, 1), i, jnp.int32)
    dnums = lax.GatherDimensionNumbers(
        offset_dims=(), collapsed_slice_dims=(0,), start_index_map=(0,))
    return lax.gather(v, idx, dnums, (1,),
                      mode=lax.GatherScatterMode.PROMISE_IN_BOUNDS)


def _deg_body(dste_hbm, ew_hbm, z_hbm, degp_hbm,
              dstb, ewb, wrows0, wrows1, wtmp, acc, sems0, sems1):
    c = lax.axis_index("c")
    s = lax.axis_index("s")
    ept = dste_hbm.shape[0] // (NC * NS)
    nchunks = ept // G
    half = nchunks // HSPLIT
    hw = half * G
    wbufs = ((wrows0, sems0), (wrows1, sems1))

    pltpu.sync_copy(z_hbm.at[pl.ds(s * RPT, RPT)], acc.at[pl.ds(s * RPT, RPT)])
    plsc.subcore_barrier()

    def build(wrows, j):
        def grp(g, _):
            w16 = ewb[pl.ds(j * G + g * 16, 16)]
            for e2 in range(16):
                wtmp[pl.ds(0, 16)] = _lane_bcast(w16, e2)
                wrows[g * 16 + e2, pl.ds(0, 16)] = wtmp[pl.ds(0, 16)]
            return 0
        lax.fori_loop(0, G // 16, grp, 0)

    def drain(wrows, sem):
        pltpu.make_async_copy(z_hbm.at[pl.ds(0, G), pl.ds(0, 32)], wrows,
                              sem).wait()

    def hloop(h, _):
        e0 = (c * NS + s) * ept + h * hw
        pltpu.sync_copy(dste_hbm.at[pl.ds(e0, hw)], dstb)
        pltpu.sync_copy(ew_hbm.at[pl.ds(e0, hw)], ewb)

        def pair(j2, _):
            for b in (0, 1):
                j = 2 * j2 + b
                wrows_c, sems_c = wbufs[b]

                @pl.when(j >= 2)
                def _():
                    drain(wrows_c, sems_c)
                build(wrows_c, j)
                pltpu.async_copy(wrows_c, acc.at[dstb.at[pl.ds(j * G, G)]],
                                 sems_c, add=True)
            return 0
        lax.fori_loop(0, half // 2, pair, 0)
        drain(wrows0, sems0)
        drain(wrows1, sems1)
        return 0
    lax.fori_loop(0, HSPLIT, hloop, 0)
    plsc.subcore_barrier()
    pltpu.sync_copy(acc.at[pl.ds(s * RPT, RPT)],
                    degp_hbm.at[c, pl.ds(s * RPT, RPT)])


def _stage_a(dst, ew):
    ept = dst.shape[0] // (NC * NS)
    hw = (ept // G // HSPLIT) * G
    mesh = plsc.VectorSubcoreMesh(core_axis_name="c", subcore_axis_name="s")
    f = functools.partial(
        pl.kernel,
        out_type=jax.ShapeDtypeStruct((NC, NPAD, 32), jnp.float32),
        mesh=mesh,
        compiler_params=pltpu.CompilerParams(use_tc_tiling_on_sc=False),
        scratch_types=[
            pltpu.VMEM((hw,), jnp.int32),
            pltpu.VMEM((hw,), jnp.float32),
            pltpu.VMEM((G, 32), jnp.float32),
            pltpu.VMEM((G, 32), jnp.float32),
            pltpu.VMEM((16,), jnp.float32),
            pltpu.VMEM_SHARED((NPAD, 32), jnp.float32),
            pltpu.SemaphoreType.DMA,
            pltpu.SemaphoreType.DMA,
        ],
    )(_deg_body)
    return f(dst, ew, jnp.zeros((NPAD, 32), jnp.float32))


def _prescale_body(degp_ref, x_ref, dinv_ref, xs_ref):
    deg = degp_ref[0, :, 0:1] + degp_ref[1, :, 0:1] + 1.0
    dv = lax.rsqrt(deg)
    dinv_ref[...] = dv
    xf = x_ref[...].reshape(B, RB, C)
    xs = dv[None, :, :] * xf
    xs_ref[0] = xs[:, :, :CH]
    xs_ref[1] = xs[:, :, CH:]


def _stage_b(degp, x4, interpret=False):
    return pl.pallas_call(
        _prescale_body,
        grid=(NB,),
        in_specs=[
            pl.BlockSpec((2, RB, 32), lambda i: (0, i, 0)),
            pl.BlockSpec((B, RB, F, T), lambda i: (0, i, 0, 0)),
        ],
        out_specs=[
            pl.BlockSpec((RB, 1), lambda i: (i, 0)),
            pl.BlockSpec((2, B, RB, CH), lambda i: (0, 0, i, 0)),
        ],
        out_shape=[
            jax.ShapeDtypeStruct((N, 1), jnp.float32),
            jax.ShapeDtypeStruct((2, B, N, CH), jnp.float32),
        ],
        interpret=interpret,
    )(degp, x4)


def _mp_body(xs_hbm, src_hbm, dst_hbm, ew_hbm, z_hbm, u_hbm,
             srcb, dstb, ewb, rows0, rows1, rows2, rows3,
             acc, semg0, semg1, sems0, sems1, semg2, semg3, sems2, sems3):
    c = lax.axis_index("c")
    s = lax.axis_index("s")
    ept = src_hbm.shape[0] // NS
    nchunks = ept // G
    half = nchunks // HSPLIT
    hw = half * G
    rbufs = ((rows0, semg0, sems0), (rows1, semg1, sems1),
             (rows2, semg2, sems2), (rows3, semg3, sems3))

    def scale(rows, j):
        def grp(g, _):
            w16 = ewb[pl.ds(j * G + g * 16, 16)]
            for e2 in range(16):
                wb = _lane_bcast(w16, e2)
                e = g * 16 + e2
                for jj in range(CH // 16):
                    rows[e, pl.ds(jj * 16, 16)] = (
                        rows[e, pl.ds(jj * 16, 16)] * wb)
            return 0
        lax.fori_loop(0, G // 16, grp, 0)

    def drain(rows, sem):
        pltpu.make_async_copy(z_hbm.at[pl.ds(0, G)], rows, sem).wait()

    def phase(p, _):
        roff = p * (B * N) + c * N
        pltpu.sync_copy(z_hbm.at[pl.ds(s * RPT, RPT)],
                        acc.at[pl.ds(s * RPT, RPT)])
        plsc.subcore_barrier()

        def hloop(h, _):
            e0 = s * ept + h * hw
            pltpu.sync_copy(src_hbm.at[pl.ds(e0, hw)], srcb)
            pltpu.sync_copy(dst_hbm.at[pl.ds(e0, hw)], dstb)
            pltpu.sync_copy(ew_hbm.at[pl.ds(e0, hw)], ewb)

            off16 = jnp.full((16,), roff, jnp.int32)

            def adj(r, _):
                srcb[pl.ds(r * 16, 16)] = srcb[pl.ds(r * 16, 16)] + off16
                return 0
            lax.fori_loop(0, hw // 16, adj, 0)

            pltpu.async_copy(xs_hbm.at[srcb.at[pl.ds(0, G)]], rows0, semg0)
            pltpu.async_copy(xs_hbm.at[srcb.at[pl.ds(G, G)]], rows1, semg1)

            def quad(j4, _):
                for b in (0, 1, 2, 3):
                    j = 4 * j4 + b
                    rows_c, semg_c, sems_c = rbufs[b]
                    rows_n, semg_n, sems_n = rbufs[(b + 2) % 4]

                    @pl.when(j >= 2)
                    def _():
                        drain(rows_n, sems_n)

                    @pl.when(j + 2 < half)
                    def _():
                        pltpu.async_copy(
                            xs_hbm.at[srcb.at[pl.ds((j + 2) * G, G)]],
                            rows_n, semg_n)

                    drain(rows_c, semg_c)
                    scale(rows_c, j)
                    pltpu.async_copy(rows_c, acc.at[dstb.at[pl.ds(j * G, G)]],
                                     sems_c, add=True)
                return 0
            lax.fori_loop(0, half // 4, quad, 0)
            drain(rows2, sems2)
            drain(rows3, sems3)
            return 0
        lax.fori_loop(0, HSPLIT, hloop, 0)
        plsc.subcore_barrier()

        @pl.when(s < NS - 1)
        def _():
            pltpu.sync_copy(acc.at[pl.ds(s * RPT, RPT)],
                            u_hbm.at[pl.ds(roff + s * RPT, RPT)])

        @pl.when(s == NS - 1)
        def _():
            last = N - (NS - 1) * RPT
            pltpu.sync_copy(acc.at[pl.ds((NS - 1) * RPT, last)],
                            u_hbm.at[pl.ds(roff + (NS - 1) * RPT, last)])
        return 0
    lax.fori_loop(0, 2, phase, 0)


def _stage_c(XsAll, srce, dste, ewe):
    ept = srce.shape[0] // NS
    hw = (ept // G // HSPLIT) * G
    mesh = plsc.VectorSubcoreMesh(core_axis_name="c", subcore_axis_name="s")
    f = functools.partial(
        pl.kernel,
        out_type=jax.ShapeDtypeStruct((2 * B * N, CH), jnp.float32),
        mesh=mesh,
        compiler_params=pltpu.CompilerParams(use_tc_tiling_on_sc=False),
        scratch_types=[
            pltpu.VMEM((hw,), jnp.int32),
            pltpu.VMEM((hw,), jnp.int32),
            pltpu.VMEM((hw,), jnp.float32),
            pltpu.VMEM((G, CH), jnp.float32),
            pltpu.VMEM((G, CH), jnp.float32),
            pltpu.VMEM((G, CH), jnp.float32),
            pltpu.VMEM((G, CH), jnp.float32),
            pltpu.VMEM_SHARED((NPAD, CH), jnp.float32),
            pltpu.SemaphoreType.DMA,
            pltpu.SemaphoreType.DMA,
            pltpu.SemaphoreType.DMA,
            pltpu.SemaphoreType.DMA,
            pltpu.SemaphoreType.DMA,
            pltpu.SemaphoreType.DMA,
            pltpu.SemaphoreType.DMA,
            pltpu.SemaphoreType.DMA,
        ],
    )(_mp_body)
    return f(XsAll, srce, dste, ewe, jnp.zeros((NPAD, CH), jnp.float32))


def _stage_d_body(xsa_ref, xsb_ref, ua_ref, ub_ref, dinv_ref, lin2_ref,
                  wz_ref, bz_ref, wh_ref, bh_ref, p_ref, linw_ref, linb_ref,
                  lin2b_ref, out_ref):
    i = pl.program_id(1)
    dv = dinv_ref[...]
    ya = dv * (xsa_ref[0] + ua_ref[0])
    yb = dv * (xsb_ref[0] + ub_ref[0])
    wz = wz_ref[...]
    wh = wh_ref[...]
    z = jax.nn.sigmoid(
        jnp.dot(ya, wz[:CH], preferred_element_type=jnp.float32)
        + jnp.dot(yb, wz[CH:], preferred_element_type=jnp.float32)
        + bz_ref[...])
    ht = jnp.tanh(
        jnp.dot(ya, wh[:CH], preferred_element_type=jnp.float32)
        + jnp.dot(yb, wh[CH:], preferred_element_type=jnp.float32)
        + bh_ref[...])
    g = (1.0 - z) * ht
    hacc = jnp.dot(g, p_ref[...], preferred_element_type=jnp.float32)
    rh = jax.nn.relu(hacc)
    h1 = jnp.dot(rh, linw_ref[...],
                 preferred_element_type=jnp.float32) + linb_ref[...]
    contrib = lax.dot_general(lin2_ref[...], h1, (((0,), (0,)), ((), ())),
                              preferred_element_type=jnp.float32)

    @pl.when(i == 0)
    def _():
        out_ref[0] = jnp.broadcast_to(lin2b_ref[...], (TARGETS, T))

    out_ref[0] += contrib


def _stage_d(XsA, XsB, UA, UB, dinv, lin2_W, WzBD, bzBD, WhBD, bhBD, P,
             lin_W, lin_b, lin2_b, interpret=False):
    grid = (B, NB)
    return pl.pallas_call(
        _stage_d_body,
        grid=grid,
        in_specs=[
            pl.BlockSpec((1, RB, CH), lambda b, i: (b, i, 0)),
            pl.BlockSpec((1, RB, CH), lambda b, i: (b, i, 0)),
            pl.BlockSpec((1, RB, CH), lambda b, i: (b, i, 0)),
            pl.BlockSpec((1, RB, CH), lambda b, i: (b, i, 0)),
            pl.BlockSpec((RB, 1), lambda b, i: (i, 0)),
            pl.BlockSpec((RB, TARGETS), lambda b, i: (i, 0)),
            pl.BlockSpec((C, T * OUT), lambda b, i: (0, 0)),
            pl.BlockSpec((1, T * OUT), lambda b, i: (0, 0)),
            pl.BlockSpec((C, T * OUT), lambda b, i: (0, 0)),
            pl.BlockSpec((1, T * OUT), lambda b, i: (0, 0)),
            pl.BlockSpec((T * OUT, OUT), lambda b, i: (0, 0)),
            pl.BlockSpec((OUT, T), lambda b, i: (0, 0)),
            pl.BlockSpec((1, T), lambda b, i: (0, 0)),
            pl.BlockSpec((TARGETS, 1), lambda b, i: (0, 0)),
        ],
        out_specs=pl.BlockSpec((1, TARGETS, T), lambda b, i: (b, 0, 0)),
        out_shape=jax.ShapeDtypeStruct((B, TARGETS, T), jnp.float32),
        interpret=interpret,
    )(XsA, XsB, UA, UB, dinv, lin2_W, WzBD, bzBD, WhBD, bhBD, P, lin_W,
      lin_b, lin2_b)


def kernel(x, edge_index, edge_weight, attention, W_z, b_z, W_r, b_r, W_h, b_h,
           Lz_W, Lz_b, Lr_W, Lr_b, Lh_W, Lh_b, lin_W, lin_b, lin2_W, lin2_b):
    src, dst = edge_index[0], edge_index[1]

    degp = _stage_a(dst, edge_weight)

    dinv, XsAll = _stage_b(degp, x)

    UAll = _stage_c(XsAll.reshape(2 * B * N, CH), src, dst,
                    edge_weight).reshape(2, B, N, CH)

    probs = jax.nn.softmax(attention)
    Wz2 = W_z @ Lz_W[:OUT]
    bz2 = b_z @ Lz_W[:OUT] + Lz_b
    Wh2 = W_h @ Lh_W[:OUT]
    bh2 = b_h @ Lh_W[:OUT] + Lh_b
    fi = jnp.arange(C) // T
    ti = jnp.arange(C) % T
    pi = jnp.arange(T * OUT) // OUT
    oi = jnp.arange(T * OUT) % OUT
    WzBD = Wz2[fi[:, None], oi[None, :]] * (ti[:, None] == pi[None, :])
    WhBD = Wh2[fi[:, None], oi[None, :]] * (ti[:, None] == pi[None, :])
    bzBD = jnp.tile(bz2, T)[None, :]
    bhBD = jnp.tile(bh2, T)[None, :]
    P = jnp.kron(probs[:, None], jnp.eye(OUT, dtype=jnp.float32))

    return _stage_d(XsAll[0], XsAll[1], UAll[0], UAll[1],
                    dinv, lin2_W, WzBD, bzBD, WhBD, bhBD, P, lin_W,
                    lin_b[None, :], lin2_b[:, None])
